# baseline (speedup 1.0000x reference)
import functools

import jax
import jax.numpy as jnp
from jax import lax
from jax.experimental import pallas as pl
from jax.experimental.pallas import tpu as pltpu
from jax.experimental.pallas import tpu_sc as plsc

D_MODEL = 1024
N_EXP = 16
D_EXPERT = 256
N_SHARED = 1
K_FFN = 2
N_ROUTED = N_EXP - N_SHARED
S = 2048
BLK = 256
NW = 16
TPW = S // NW
PPW = 2 * TPW
P = 2 * S
GBLK = 128
NBLK = P // GBLK + N_ROUTED
PAD = NBLK * GBLK
L = 16


def _route_shared_kernel(x_ref, sel_ref, est_ref, k_ref, v_ref,
                         base_ref, idx_ref, i1_ref, i2_ref, w_ref):
    logits = jnp.dot(sel_ref[...], est_ref[...],
                     preferred_element_type=jnp.float32)
    aff = jax.nn.sigmoid(logits)
    ids = lax.broadcasted_iota(jnp.int32, (BLK, N_EXP), 1)
    neg = jnp.where(ids < N_ROUTED, aff, -jnp.inf)
    m1 = jnp.max(neg, axis=1, keepdims=True)
    i1 = jnp.min(jnp.where(neg == m1, ids, N_EXP), axis=1, keepdims=True)
    neg2 = jnp.where(ids == i1, -jnp.inf, neg)
    m2 = jnp.max(neg2, axis=1, keepdims=True)
    i2 = jnp.min(jnp.where(neg2 == m2, ids, N_EXP), axis=1, keepdims=True)
    shared = jnp.full((BLK, 1), N_ROUTED, dtype=jnp.int32)
    idx_ref[...] = jnp.concatenate([i1, i2, shared], axis=1)
    zpad_i = jnp.zeros((BLK, 7), jnp.int32)
    i1_ref[...] = jnp.concatenate([i1, zpad_i], axis=1)
    i2_ref[...] = jnp.concatenate([i2, zpad_i], axis=1)
    zpad_f = jnp.zeros((BLK, 6), jnp.float32)
    w_ref[...] = jnp.concatenate([m1, m2, zpad_f], axis=1)

    h = jnp.dot(x_ref[...], k_ref[0], preferred_element_type=jnp.float32)
    h = h * jax.nn.sigmoid(h)
    hw = h * aff[:, N_ROUTED:N_ROUTED + 1]
    base_ref[...] = jnp.dot(hw, v_ref[0], preferred_element_type=jnp.float32)


def _route_shared(x, sel, est, keys_w, values_w):
    return pl.pallas_call(
        _route_shared_kernel,
        grid=(S // BLK,),
        in_specs=[
            pl.BlockSpec((BLK, D_MODEL), lambda t: (t, 0)),
            pl.BlockSpec((BLK, D_MODEL), lambda t: (t, 0)),
            pl.BlockSpec((D_MODEL, N_EXP), lambda t: (0, 0)),
            pl.BlockSpec((1, D_MODEL, D_EXPERT), lambda t: (N_ROUTED, 0, 0)),
            pl.BlockSpec((1, D_EXPERT, D_MODEL), lambda t: (N_ROUTED, 0, 0)),
        ],
        out_specs=[
            pl.BlockSpec((BLK, D_MODEL), lambda t: (t, 0)),
            pl.BlockSpec((BLK, 3), lambda t: (t, 0)),
            pl.BlockSpec((BLK, 8), lambda t: (t, 0)),
            pl.BlockSpec((BLK, 8), lambda t: (t, 0)),
            pl.BlockSpec((BLK, 8), lambda t: (t, 0)),
        ],
        out_shape=[
            jax.ShapeDtypeStruct((S, D_MODEL), jnp.float32),
            jax.ShapeDtypeStruct((S, 3), jnp.int32),
            jax.ShapeDtypeStruct((S, 8), jnp.int32),
            jax.ShapeDtypeStruct((S, 8), jnp.int32),
            jax.ShapeDtypeStruct((S, 8), jnp.float32),
        ],
        compiler_params=pltpu.CompilerParams(
            dimension_semantics=("arbitrary",),
        ),
    )(x, sel, est, keys_w, values_w)


def _bincount16(idv, lane):
    c = jnp.zeros((L,), jnp.int32)
    for e in range(N_ROUTED):
        n = plsc.all_reduce_population_count(idv == e)
        c = c + jnp.where(lane == e, n, 0)
    return c


def _dispatch_body(x_hbm, i1_hbm, i2_hbm,
                   xg_hbm, pos_hbm, blk_hbm,
                   i1sl, i2sl, cnt, h32, rc, posb, tok, pchunk, rows, beb,
                   hist_sh):
    w = lax.axis_index("s")
    lane = lax.iota(jnp.int32, L)

    pltpu.sync_copy(i1_hbm.at[pl.ds(w * TPW * 8, TPW * 8)], i1sl)
    pltpu.sync_copy(i2_hbm.at[pl.ds(w * TPW * 8, TPW * 8)], i2sl)

    idvs = []
    counts = []
    ctot = jnp.zeros((L,), jnp.int32)
    half_vregs = TPW // L
    for k in range(PPW // L):
        jv = (k % half_vregs) * L + lane
        idsrc = i1sl if k < half_vregs else i2sl
        idv = plsc.load_gather(idsrc, [jv * 8])
        c = _bincount16(idv, lane)
        idvs.append(idv)
        counts.append(c)
        ctot = ctot + c
    cnt[...] = ctot

    pltpu.sync_copy(cnt, hist_sh.at[pl.ds(w * L, L)])
    plsc.subcore_barrier()
    pltpu.sync_copy(hist_sh, h32)

    c_tot = jnp.zeros((L,), jnp.int32)
    base_partial = jnp.zeros((L,), jnp.int32)
    for j in range(NW):
        hj = h32[pl.ds(j * L, L)]
        c_tot = c_tot + hj
        flag = jnp.where(j < w, 1, 0).astype(jnp.int32)
        base_partial = base_partial + hj * flag
    padded = ((c_tot + (GBLK - 1)) >> 7) << 7
    incl = plsc.cumsum(padded)
    off = incl - padded
    rc[...] = off + base_partial
    off_end = off + padded

    @pl.when(w == 0)
    def _emit_block_experts():
        for t3 in range(3):
            bv = lane + t3 * L
            be = jnp.zeros((L,), jnp.int32)
            for e in range(N_ROUTED):
                be = be + jnp.where(bv * GBLK >= off_end[e], 1, 0)
            beb[pl.ds(t3 * L, L)] = be
        pltpu.sync_copy(beb, blk_hbm)

    for k in range(PPW // L):
        idv = idvs[k]
        r = jnp.zeros((L,), jnp.int32)
        for l in range(L):
            idl = idv[l]
            r = r + jnp.where((idv == idl) & (lane > l), 1, 0)
        base = plsc.load_gather(rc, [idv])
        posb[pl.ds(k * L, L)] = base + r
        rc[...] = rc[...] + counts[k]
    pltpu.sync_copy(posb, pos_hbm.at[pl.ds(w * PPW, PPW)])

    for ch in range(PPW // 32):
        for half in range(2):
            j0 = ch * 32 + half * L
            tokv = (w * TPW) + ((j0 + lane) % TPW)
            tok[pl.ds(half * L, L)] = tokv
            pchunk[pl.ds(half * L, L)] = posb[pl.ds(j0, L)]
        pltpu.sync_copy(x_hbm.at[tok], rows)
        pltpu.sync_copy(rows, xg_hbm.at[pchunk])


def _dispatch(x, i1c, i2c):
    mesh = plsc.VectorSubcoreMesh(core_axis_name="c", subcore_axis_name="s",
                                  num_cores=1, num_subcores=16)
    f = pl.kernel(
        _dispatch_body,
        out_type=[
            jax.ShapeDtypeStruct((PAD, D_MODEL), jnp.float32),
            jax.ShapeDtypeStruct((P,), jnp.int32),
            jax.ShapeDtypeStruct((NBLK + 1,), jnp.int32),
        ],
        mesh=mesh,
        scratch_types=[
            pltpu.VMEM((TPW * 8,), jnp.int32),
            pltpu.VMEM((TPW * 8,), jnp.int32),
            pltpu.VMEM((L,), jnp.int32),
            pltpu.VMEM((NW * L,), jnp.int32),
            pltpu.VMEM((L,), jnp.int32),
            pltpu.VMEM((PPW,), jnp.int32),
            pltpu.VMEM((32,), jnp.int32),
            pltpu.VMEM((32,), jnp.int32),
            pltpu.VMEM((32, D_MODEL), jnp.float32),
            pltpu.VMEM((NBLK + 1,), jnp.int32),
            pltpu.VMEM_SHARED((NW * L,), jnp.int32),
        ],
        compiler_params=pltpu.CompilerParams(needs_layout_passes=False),
    )
    return f(x, i1c, i2c)


def _ggemm_kernel(be_ref, xg_ref, k_ref, v_ref, o_ref):
    h = jnp.dot(xg_ref[...], k_ref[0], preferred_element_type=jnp.float32)
    h = h * jax.nn.sigmoid(h)
    o_ref[...] = jnp.dot(h, v_ref[0], preferred_element_type=jnp.float32)


def _ggemm(blk_e, xg, keys_w, values_w):
    return pl.pallas_call(
        _ggemm_kernel,
        grid_spec=pltpu.PrefetchScalarGridSpec(
            num_scalar_prefetch=1,
            grid=(NBLK,),
            in_specs=[
                pl.BlockSpec((GBLK, D_MODEL), lambda b, be: (b, 0)),
                pl.BlockSpec((1, D_MODEL, D_EXPERT),
                             lambda b, be: (be[b], 0, 0)),
                pl.BlockSpec((1, D_EXPERT, D_MODEL),
                             lambda b, be: (be[b], 0, 0)),
            ],
            out_specs=pl.BlockSpec((GBLK, D_MODEL), lambda b, be: (b, 0)),
        ),
        out_shape=jax.ShapeDtypeStruct((PAD, D_MODEL), jnp.float32),
        compiler_params=pltpu.CompilerParams(
            dimension_semantics=("arbitrary",),
        ),
    )(blk_e, xg, keys_w, values_w)


def _combine_body(base_hbm, os_hbm, pos_hbm, w_hbm, out_hbm,
                  posb, wsl, bbuf, g0, g1, idx0, idx1):
    w = lax.axis_index("s")
    lane = lax.iota(jnp.int32, L)

    pltpu.sync_copy(pos_hbm.at[pl.ds(w * PPW, PPW)], posb)
    pltpu.sync_copy(w_hbm.at[pl.ds(w * TPW * 8, TPW * 8)], wsl)

    for ch in range(TPW // L):
        jv = ch * L + lane
        idx0[...] = plsc.load_gather(posb, [jv])
        idx1[...] = plsc.load_gather(posb, [TPW + jv])
        pltpu.sync_copy(os_hbm.at[idx0], g0)
        pltpu.sync_copy(os_hbm.at[idx1], g1)
        pltpu.sync_copy(base_hbm.at[pl.ds(w * TPW + ch * L, L)], bbuf)

        def _token(j, _):
            jj = jnp.full((L,), 8 * (ch * L + j), jnp.int32)
            w0 = plsc.load_gather(wsl, [jj])
            w1 = plsc.load_gather(wsl, [jj + 1])

            def _slice(r, __):
                sl = pl.ds(r * L, L)
                acc = bbuf[j, sl] + w0 * g0[j, sl] + w1 * g1[j, sl]
                bbuf[j, sl] = acc
                return __

            lax.fori_loop(0, D_MODEL // L, _slice, 0, unroll=4)
            return _

        lax.fori_loop(0, L, _token, 0)
        pltpu.sync_copy(bbuf, out_hbm.at[pl.ds(w * TPW + ch * L, L)])


def _combine(out_base, os, pos, waff):
    mesh = plsc.VectorSubcoreMesh(core_axis_name="c", subcore_axis_name="s",
                                  num_cores=1, num_subcores=16)
    f = pl.kernel(
        _combine_body,
        out_type=jax.ShapeDtypeStruct((S, D_MODEL), jnp.float32),
        mesh=mesh,
        scratch_types=[
            pltpu.VMEM((PPW,), jnp.int32),
            pltpu.VMEM((TPW * 8,), jnp.float32),
            pltpu.VMEM((L, D_MODEL), jnp.float32),
            pltpu.VMEM((L, D_MODEL), jnp.float32),
            pltpu.VMEM((L, D_MODEL), jnp.float32),
            pltpu.VMEM((L,), jnp.int32),
            pltpu.VMEM((L,), jnp.int32),
        ],
        compiler_params=pltpu.CompilerParams(needs_layout_passes=False),
    )
    return f(out_base, os, pos, waff)


@jax.jit
def kernel(token_stream, selection_input, keys_w, values_w, expert_sel):
    x = token_stream.reshape(S, D_MODEL)
    sel = selection_input.reshape(S, D_MODEL)
    est = expert_sel.T

    out_base, sel_idx, i1c, i2c, waff = _route_shared(
        x, sel, est, keys_w, values_w)
    xg, pos, blk_e = _dispatch(x, i1c.reshape(S * 8), i2c.reshape(S * 8))
    os = _ggemm(blk_e, xg, keys_w, values_w)
    out = _combine(out_base, os, pos, waff.reshape(S * 8))

    return out.reshape(1, S, D_MODEL), sel_idx.reshape(1, S, 3)


# speedup vs baseline: 1.2043x; 1.2043x over previous
import functools

import jax
import jax.numpy as jnp
from jax import lax
from jax.experimental import pallas as pl
from jax.experimental.pallas import tpu as pltpu
from jax.experimental.pallas import tpu_sc as plsc

D_MODEL = 1024
N_EXP = 16
D_EXPERT = 256
N_SHARED = 1
K_FFN = 2
N_ROUTED = N_EXP - N_SHARED
S = 2048
BLK = 256
NW = 16
TPW = S // NW
PPW = 2 * TPW
P = 2 * S
GBLK = 128
NBLK = P // GBLK + N_ROUTED
PAD = NBLK * GBLK
L = 16


def _route_shared_kernel(x_ref, sel_ref, est_ref, k_ref, v_ref,
                         base_ref, idx_ref, i1_ref, i2_ref, w_ref):
    logits = jnp.dot(sel_ref[...], est_ref[...],
                     preferred_element_type=jnp.float32)
    aff = jax.nn.sigmoid(logits)
    ids = lax.broadcasted_iota(jnp.int32, (BLK, N_EXP), 1)
    neg = jnp.where(ids < N_ROUTED, aff, -jnp.inf)
    m1 = jnp.max(neg, axis=1, keepdims=True)
    i1 = jnp.min(jnp.where(neg == m1, ids, N_EXP), axis=1, keepdims=True)
    neg2 = jnp.where(ids == i1, -jnp.inf, neg)
    m2 = jnp.max(neg2, axis=1, keepdims=True)
    i2 = jnp.min(jnp.where(neg2 == m2, ids, N_EXP), axis=1, keepdims=True)
    shared = jnp.full((BLK, 1), N_ROUTED, dtype=jnp.int32)
    idx_ref[...] = jnp.concatenate([i1, i2, shared], axis=1)
    zpad_i = jnp.zeros((BLK, 7), jnp.int32)
    i1_ref[...] = jnp.concatenate([i1, zpad_i], axis=1)
    i2_ref[...] = jnp.concatenate([i2, zpad_i], axis=1)
    zpad_f = jnp.zeros((BLK, 6), jnp.float32)
    w_ref[...] = jnp.concatenate([m1, m2, zpad_f], axis=1)

    h = jnp.dot(x_ref[...], k_ref[0], preferred_element_type=jnp.float32)
    h = h * jax.nn.sigmoid(h)
    hw = h * aff[:, N_ROUTED:N_ROUTED + 1]
    base_ref[...] = jnp.dot(hw, v_ref[0], preferred_element_type=jnp.float32)


def _route_shared(x, sel, est, keys_w, values_w):
    return pl.pallas_call(
        _route_shared_kernel,
        grid=(S // BLK,),
        in_specs=[
            pl.BlockSpec((BLK, D_MODEL), lambda t: (t, 0)),
            pl.BlockSpec((BLK, D_MODEL), lambda t: (t, 0)),
            pl.BlockSpec((D_MODEL, N_EXP), lambda t: (0, 0)),
            pl.BlockSpec((1, D_MODEL, D_EXPERT), lambda t: (N_ROUTED, 0, 0)),
            pl.BlockSpec((1, D_EXPERT, D_MODEL), lambda t: (N_ROUTED, 0, 0)),
        ],
        out_specs=[
            pl.BlockSpec((BLK, D_MODEL), lambda t: (t, 0)),
            pl.BlockSpec((BLK, 3), lambda t: (t, 0)),
            pl.BlockSpec((BLK, 8), lambda t: (t, 0)),
            pl.BlockSpec((BLK, 8), lambda t: (t, 0)),
            pl.BlockSpec((BLK, 8), lambda t: (t, 0)),
        ],
        out_shape=[
            jax.ShapeDtypeStruct((S, D_MODEL), jnp.float32),
            jax.ShapeDtypeStruct((S, 3), jnp.int32),
            jax.ShapeDtypeStruct((S, 8), jnp.int32),
            jax.ShapeDtypeStruct((S, 8), jnp.int32),
            jax.ShapeDtypeStruct((S, 8), jnp.float32),
        ],
        compiler_params=pltpu.CompilerParams(
            dimension_semantics=("arbitrary",),
        ),
    )(x, sel, est, keys_w, values_w)


def _bincount16(idv, lane):
    c = jnp.zeros((L,), jnp.int32)
    for e in range(N_ROUTED):
        n = plsc.all_reduce_population_count(idv == e)
        c = c + jnp.where(lane == e, n, 0)
    return c


def _dispatch_body(x_hbm, i1_hbm, i2_hbm,
                   xg_hbm, pos_hbm, blk_hbm,
                   i1sl, i2sl, cnt, h32, rc, posb, tok, pchunk, rows,
                   tok2, pchunk2, rows2, gsem, ssem, beb,
                   hist_sh):
    w = lax.axis_index("s")
    lane = lax.iota(jnp.int32, L)

    pltpu.sync_copy(i1_hbm.at[pl.ds(w * TPW * 8, TPW * 8)], i1sl)
    pltpu.sync_copy(i2_hbm.at[pl.ds(w * TPW * 8, TPW * 8)], i2sl)

    idvs = []
    counts = []
    ctot = jnp.zeros((L,), jnp.int32)
    half_vregs = TPW // L
    for k in range(PPW // L):
        jv = (k % half_vregs) * L + lane
        idsrc = i1sl if k < half_vregs else i2sl
        idv = plsc.load_gather(idsrc, [jv * 8])
        c = _bincount16(idv, lane)
        idvs.append(idv)
        counts.append(c)
        ctot = ctot + c
    cnt[...] = ctot

    pltpu.sync_copy(cnt, hist_sh.at[pl.ds(w * L, L)])
    plsc.subcore_barrier()
    pltpu.sync_copy(hist_sh, h32)

    c_tot = jnp.zeros((L,), jnp.int32)
    base_partial = jnp.zeros((L,), jnp.int32)
    for j in range(NW):
        hj = h32[pl.ds(j * L, L)]
        c_tot = c_tot + hj
        flag = jnp.where(j < w, 1, 0).astype(jnp.int32)
        base_partial = base_partial + hj * flag
    padded = ((c_tot + (GBLK - 1)) >> 7) << 7
    incl = plsc.cumsum(padded)
    off = incl - padded
    rc[...] = off + base_partial
    off_end = off + padded

    @pl.when(w == 0)
    def _emit_block_experts():
        for t3 in range(3):
            bv = lane + t3 * L
            be = jnp.zeros((L,), jnp.int32)
            for e in range(N_ROUTED):
                be = be + jnp.where(bv * GBLK >= off_end[e], 1, 0)
            beb[pl.ds(t3 * L, L)] = be
        pltpu.sync_copy(beb, blk_hbm)

    for k in range(PPW // L):
        idv = idvs[k]
        r = jnp.zeros((L,), jnp.int32)
        for l in range(L):
            idl = idv[l]
            r = r + jnp.where((idv == idl) & (lane > l), 1, 0)
        base = plsc.load_gather(rc, [idv])
        posb[pl.ds(k * L, L)] = base + r
        rc[...] = rc[...] + counts[k]
    pltpu.sync_copy(posb, pos_hbm.at[pl.ds(w * PPW, PPW)])

    nch = PPW // 32
    toks = [tok, tok2]
    pchs = [pchunk, pchunk2]
    rbufs = [rows, rows2]
    gd = [None, None]
    sd = [None, None]

    def _fill_idx(ch):
        slot = ch % 2
        for half in range(2):
            j0 = ch * 32 + half * L
            tokv = (w * TPW) + ((j0 + lane) % TPW)
            toks[slot][pl.ds(half * L, L)] = tokv
            pchs[slot][pl.ds(half * L, L)] = posb[pl.ds(j0, L)]

    _fill_idx(0)
    gd[0] = pltpu.async_copy(x_hbm.at[toks[0]], rbufs[0], gsem.at[0])
    for ch in range(nch):
        slot = ch % 2
        other = 1 - slot
        if ch + 1 < nch:
            if sd[other] is not None:
                sd[other].wait()
            _fill_idx(ch + 1)
            gd[other] = pltpu.async_copy(
                x_hbm.at[toks[other]], rbufs[other], gsem.at[other])
        gd[slot].wait()
        sd[slot] = pltpu.async_copy(
            rbufs[slot], xg_hbm.at[pchs[slot]], ssem.at[slot])
    sd[0].wait()
    sd[1].wait()


def _dispatch(x, i1c, i2c):
    mesh = plsc.VectorSubcoreMesh(core_axis_name="c", subcore_axis_name="s",
                                  num_cores=1, num_subcores=16)
    f = pl.kernel(
        _dispatch_body,
        out_type=[
            jax.ShapeDtypeStruct((PAD, D_MODEL), jnp.float32),
            jax.ShapeDtypeStruct((P,), jnp.int32),
            jax.ShapeDtypeStruct((NBLK + 1,), jnp.int32),
        ],
        mesh=mesh,
        scratch_types=[
            pltpu.VMEM((TPW * 8,), jnp.int32),
            pltpu.VMEM((TPW * 8,), jnp.int32),
            pltpu.VMEM((L,), jnp.int32),
            pltpu.VMEM((NW * L,), jnp.int32),
            pltpu.VMEM((L,), jnp.int32),
            pltpu.VMEM((PPW,), jnp.int32),
            pltpu.VMEM((32,), jnp.int32),
            pltpu.VMEM((32,), jnp.int32),
            pltpu.VMEM((32, D_MODEL), jnp.float32),
            pltpu.VMEM((32,), jnp.int32),
            pltpu.VMEM((32,), jnp.int32),
            pltpu.VMEM((32, D_MODEL), jnp.float32),
            pltpu.SemaphoreType.DMA((2,)),
            pltpu.SemaphoreType.DMA((2,)),
            pltpu.VMEM((NBLK + 1,), jnp.int32),
            pltpu.VMEM_SHARED((NW * L,), jnp.int32),
        ],
        compiler_params=pltpu.CompilerParams(needs_layout_passes=False),
    )
    return f(x, i1c, i2c)


def _ggemm_kernel(be_ref, xg_ref, k_ref, v_ref, o_ref):
    h = jnp.dot(xg_ref[...], k_ref[0], preferred_element_type=jnp.float32)
    h = h * jax.nn.sigmoid(h)
    o_ref[...] = jnp.dot(h, v_ref[0], preferred_element_type=jnp.float32)


def _ggemm(blk_e, xg, keys_w, values_w):
    return pl.pallas_call(
        _ggemm_kernel,
        grid_spec=pltpu.PrefetchScalarGridSpec(
            num_scalar_prefetch=1,
            grid=(NBLK,),
            in_specs=[
                pl.BlockSpec((GBLK, D_MODEL), lambda b, be: (b, 0)),
                pl.BlockSpec((1, D_MODEL, D_EXPERT),
                             lambda b, be: (be[b], 0, 0)),
                pl.BlockSpec((1, D_EXPERT, D_MODEL),
                             lambda b, be: (be[b], 0, 0)),
            ],
            out_specs=pl.BlockSpec((GBLK, D_MODEL), lambda b, be: (b, 0)),
        ),
        out_shape=jax.ShapeDtypeStruct((PAD, D_MODEL), jnp.float32),
        compiler_params=pltpu.CompilerParams(
            dimension_semantics=("arbitrary",),
        ),
    )(blk_e, xg, keys_w, values_w)


TPW4 = S // 32
PPW4 = 2 * TPW4


def _combine_body(base_hbm, os_hbm, pos_hbm, w_hbm, out_hbm,
                  posb, wsl, bbuf, g0, g1, idx0, idx1):
    w = lax.axis_index("s") * 2 + lax.axis_index("c")
    lane = lax.iota(jnp.int32, L)

    pbase = (w // 2) * 256 + (w % 2) * 64
    pltpu.sync_copy(pos_hbm.at[pl.ds(pbase, TPW4)], posb.at[pl.ds(0, TPW4)])
    pltpu.sync_copy(pos_hbm.at[pl.ds(pbase + 128, TPW4)],
                    posb.at[pl.ds(TPW4, TPW4)])
    pltpu.sync_copy(w_hbm.at[pl.ds(w * TPW4 * 8, TPW4 * 8)], wsl)

    for ch in range(TPW4 // L):
        jv = ch * L + lane
        idx0[...] = plsc.load_gather(posb, [jv])
        idx1[...] = plsc.load_gather(posb, [TPW4 + jv])
        pltpu.sync_copy(os_hbm.at[idx0], g0)
        pltpu.sync_copy(os_hbm.at[idx1], g1)
        pltpu.sync_copy(base_hbm.at[pl.ds(w * TPW4 + ch * L, L)], bbuf)

        def _token(j, _):
            jj = jnp.full((L,), 8 * (ch * L + j), jnp.int32)
            w0 = plsc.load_gather(wsl, [jj])
            w1 = plsc.load_gather(wsl, [jj + 1])

            def _slice(r, __):
                sl = pl.ds(r * L, L)
                acc = bbuf[j, sl] + w0 * g0[j, sl] + w1 * g1[j, sl]
                bbuf[j, sl] = acc
                return __

            lax.fori_loop(0, D_MODEL // L, _slice, 0, unroll=4)
            return _

        lax.fori_loop(0, L, _token, 0)
        pltpu.sync_copy(bbuf, out_hbm.at[pl.ds(w * TPW4 + ch * L, L)])


def _combine(out_base, os, pos, waff):
    mesh = plsc.VectorSubcoreMesh(core_axis_name="c", subcore_axis_name="s",
                                  num_cores=2, num_subcores=16)
    f = pl.kernel(
        _combine_body,
        out_type=jax.ShapeDtypeStruct((S, D_MODEL), jnp.float32),
        mesh=mesh,
        scratch_types=[
            pltpu.VMEM((PPW4,), jnp.int32),
            pltpu.VMEM((TPW4 * 8,), jnp.float32),
            pltpu.VMEM((L, D_MODEL), jnp.float32),
            pltpu.VMEM((L, D_MODEL), jnp.float32),
            pltpu.VMEM((L, D_MODEL), jnp.float32),
            pltpu.VMEM((L,), jnp.int32),
            pltpu.VMEM((L,), jnp.int32),
        ],
        compiler_params=pltpu.CompilerParams(needs_layout_passes=False),
    )
    return f(out_base, os, pos, waff)


@jax.jit
def kernel(token_stream, selection_input, keys_w, values_w, expert_sel):
    x = token_stream.reshape(S, D_MODEL)
    sel = selection_input.reshape(S, D_MODEL)
    est = expert_sel.T

    out_base, sel_idx, i1c, i2c, waff = _route_shared(
        x, sel, est, keys_w, values_w)
    xg, pos, blk_e = _dispatch(x, i1c.reshape(S * 8), i2c.reshape(S * 8))
    os = _ggemm(blk_e, xg, keys_w, values_w)
    out = _combine(out_base, os, pos, waff.reshape(S * 8))

    return out.reshape(1, S, D_MODEL), sel_idx.reshape(1, S, 3)


# speedup vs baseline: 2.9167x; 2.4220x over previous
import jax
import jax.numpy as jnp
from jax.experimental import pallas as pl
from jax.experimental.pallas import tpu as pltpu

D_MODEL = 1024
N_EXP = 16
D_EXPERT = 256
N_SHARED = 1
K_FFN = 2
N_ROUTED = N_EXP - N_SHARED
S = 2048
BLK = 256


def _moe_kernel(x_ref, sel_ref, est_ref, k_ref, v_ref, out_ref, idx_ref):
    logits = jnp.dot(sel_ref[...], est_ref[...],
                     preferred_element_type=jnp.float32)
    aff = jax.nn.sigmoid(logits)
    ids = jax.lax.broadcasted_iota(jnp.int32, (BLK, N_EXP), 1)
    neg = jnp.where(ids < N_ROUTED, aff, -jnp.inf)
    m1 = jnp.max(neg, axis=1, keepdims=True)
    i1 = jnp.min(jnp.where(neg == m1, ids, N_EXP), axis=1, keepdims=True)
    neg2 = jnp.where(ids == i1, -jnp.inf, neg)
    m2 = jnp.max(neg2, axis=1, keepdims=True)
    i2 = jnp.min(jnp.where(neg2 == m2, ids, N_EXP), axis=1, keepdims=True)
    shared = jnp.full((BLK, 1), N_ROUTED, dtype=jnp.int32)
    idx_ref[...] = jnp.concatenate([i1, i2, shared], axis=1)
    selmask = (ids == i1) | (ids == i2) | (ids >= N_ROUTED)
    w = jnp.where(selmask, aff, 0.0)

    x = x_ref[...]
    acc = jnp.zeros((BLK, D_MODEL), dtype=jnp.float32)
    for e in range(N_EXP):
        h = jnp.dot(x, k_ref[e], preferred_element_type=jnp.float32)
        h = h * jax.nn.sigmoid(h)
        hw = h * w[:, e:e + 1]
        acc = acc + jnp.dot(hw, v_ref[e], preferred_element_type=jnp.float32)
    out_ref[...] = acc


@jax.jit
def kernel(token_stream, selection_input, keys_w, values_w, expert_sel):
    x = token_stream.reshape(S, D_MODEL)
    sel = selection_input.reshape(S, D_MODEL)
    est = expert_sel.T

    out, sel_idx = pl.pallas_call(
        _moe_kernel,
        grid=(S // BLK,),
        in_specs=[
            pl.BlockSpec((BLK, D_MODEL), lambda t: (t, 0)),
            pl.BlockSpec((BLK, D_MODEL), lambda t: (t, 0)),
            pl.BlockSpec((D_MODEL, N_EXP), lambda t: (0, 0)),
            pl.BlockSpec((N_EXP, D_MODEL, D_EXPERT), lambda t: (0, 0, 0)),
            pl.BlockSpec((N_EXP, D_EXPERT, D_MODEL), lambda t: (0, 0, 0)),
        ],
        out_specs=[
            pl.BlockSpec((BLK, D_MODEL), lambda t: (t, 0)),
            pl.BlockSpec((BLK, 3), lambda t: (t, 0)),
        ],
        out_shape=[
            jax.ShapeDtypeStruct((S, D_MODEL), jnp.float32),
            jax.ShapeDtypeStruct((S, 3), jnp.int32),
        ],
        compiler_params=pltpu.CompilerParams(
            dimension_semantics=("arbitrary",),
        ),
    )(x, sel, est, keys_w, values_w)

    return out.reshape(1, S, D_MODEL), sel_idx.reshape(1, S, 3)
